# baseline (speedup 1.0000x reference)
import jax
import jax.numpy as jnp
from jax import lax
from jax.experimental import pallas as pl
from jax.experimental.pallas import tpu as pltpu

N_FACTORS = 64
BATCH = 16384
BLOCK = 1024
GRID = BATCH // BLOCK


NSEM = 8


def _body(uidx, iidx, ut_hbm, it_hbm, out_ref, ubuf, ibuf, sems):
    def fire(r, _):
        q = r % NSEM
        pltpu.async_copy(ut_hbm.at[uidx[r]], ubuf.at[r], sems.at[q])
        pltpu.async_copy(it_hbm.at[iidx[r]], ibuf.at[r], sems.at[NSEM + q])
        return 0

    lax.fori_loop(0, BLOCK, fire, 0, unroll=8)

    per = BLOCK // NSEM
    for q in range(NSEM):
        pltpu.make_async_copy(
            ut_hbm.at[pl.ds(0, per)], ubuf.at[pl.ds(0, per)], sems.at[q]
        ).wait()
        pltpu.make_async_copy(
            it_hbm.at[pl.ds(0, per)], ibuf.at[pl.ds(0, per)], sems.at[NSEM + q]
        ).wait()

    out_ref[...] = jnp.sum(ubuf[...] * ibuf[...], axis=1)


@jax.jit
def _mf(users, items, user_table, item_table):
    f = pl.pallas_call(
        _body,
        grid=(GRID,),
        in_specs=[
            pl.BlockSpec((BLOCK,), lambda b: (b,), memory_space=pltpu.SMEM),
            pl.BlockSpec((BLOCK,), lambda b: (b,), memory_space=pltpu.SMEM),
            pl.BlockSpec(memory_space=pltpu.HBM),
            pl.BlockSpec(memory_space=pltpu.HBM),
        ],
        out_specs=pl.BlockSpec((BLOCK,), lambda b: (b,)),
        out_shape=jax.ShapeDtypeStruct((BATCH,), jnp.float32),
        scratch_shapes=[
            pltpu.VMEM((BLOCK, N_FACTORS), jnp.float32),
            pltpu.VMEM((BLOCK, N_FACTORS), jnp.float32),
            pltpu.SemaphoreType.DMA((2 * NSEM,)),
        ],
    )
    return f(users, items, user_table, item_table)


def kernel(users, items, user_table, item_table):
    return _mf(users, items, user_table, item_table)


# speedup vs baseline: 1.0128x; 1.0128x over previous
import jax
import jax.numpy as jnp
from jax import lax
from jax.experimental import pallas as pl
from jax.experimental.pallas import tpu as pltpu

N_FACTORS = 64
BATCH = 16384
BLOCK = 1024
GRID = BATCH // BLOCK


NSEM = 8


def _body(uidx, iidx, ut_hbm, it_hbm, out_ref, ubuf, ibuf, sems):
    def fire(r, _):
        q = r % NSEM
        pltpu.async_copy(ut_hbm.at[uidx[r]], ubuf.at[r], sems.at[q])
        pltpu.async_copy(
            it_hbm.at[iidx[r]], ibuf.at[r], sems.at[NSEM + q], priority=1
        )
        return 0

    lax.fori_loop(0, BLOCK, fire, 0, unroll=8)

    per = BLOCK // NSEM
    for q in range(NSEM):
        pltpu.make_async_copy(
            ut_hbm.at[pl.ds(0, per)], ubuf.at[pl.ds(0, per)], sems.at[q]
        ).wait()
        pltpu.make_async_copy(
            it_hbm.at[pl.ds(0, per)], ibuf.at[pl.ds(0, per)], sems.at[NSEM + q]
        ).wait()

    out_ref[...] = jnp.sum(ubuf[...] * ibuf[...], axis=1)


@jax.jit
def _mf(users, items, user_table, item_table):
    f = pl.pallas_call(
        _body,
        grid=(GRID,),
        in_specs=[
            pl.BlockSpec((BLOCK,), lambda b: (b,), memory_space=pltpu.SMEM),
            pl.BlockSpec((BLOCK,), lambda b: (b,), memory_space=pltpu.SMEM),
            pl.BlockSpec(memory_space=pltpu.HBM),
            pl.BlockSpec(memory_space=pltpu.HBM),
        ],
        out_specs=pl.BlockSpec((BLOCK,), lambda b: (b,)),
        out_shape=jax.ShapeDtypeStruct((BATCH,), jnp.float32),
        scratch_shapes=[
            pltpu.VMEM((BLOCK, N_FACTORS), jnp.float32),
            pltpu.VMEM((BLOCK, N_FACTORS), jnp.float32),
            pltpu.SemaphoreType.DMA((2 * NSEM,)),
        ],
    )
    return f(users, items, user_table, item_table)


def kernel(users, items, user_table, item_table):
    return _mf(users, items, user_table, item_table)


# speedup vs baseline: 1.1711x; 1.1563x over previous
import jax
import jax.numpy as jnp
from jax import lax
from jax.experimental import pallas as pl
from jax.experimental.pallas import tpu as pltpu
from jax.experimental.pallas import tpu_sc as plsc

N_FACTORS = 64
BATCH = 16384
NW = 32
B_PER_W = BATCH // NW
HALF = B_PER_W // 2


def _body(users_hbm, items_hbm, ut_hbm, it_hbm, out_hbm,
          uidx, iidx, urows, irows, out_v, sem):
    wid = lax.axis_index("s") * 2 + lax.axis_index("c")

    pltpu.sync_copy(users_hbm.at[wid], uidx.at[pl.ds(0, B_PER_W)])
    pltpu.sync_copy(items_hbm.at[wid], iidx.at[pl.ds(0, B_PER_W)])

    iota = lax.iota(jnp.int32, 16)

    for half in range(2):
        def fire(t, _, half=half):
            t2 = half * (HALF // 8) + t
            uvec = uidx[pl.ds(t2 * 8, 16)]
            ivec = iidx[pl.ds(t2 * 8, 16)]
            for l in range(8):
                slot = t * 8 + l
                pltpu.async_copy(ut_hbm.at[uvec[l]], urows.at[slot], sem)
                pltpu.async_copy(it_hbm.at[ivec[l]], irows.at[slot], sem)
            return 0

        lax.fori_loop(0, HALF // 8, fire, 0)

        pltpu.make_async_copy(ut_hbm.at[pl.ds(0, HALF)], urows, sem).wait()
        pltpu.make_async_copy(it_hbm.at[pl.ds(0, HALF)], irows, sem).wait()

        def group(g, _, half=half):
            slotv = g * 16 + iota
            acc = jnp.zeros((16,), jnp.float32)
            for d in range(N_FACTORS):
                colv = (iota + d) & (N_FACTORS - 1)
                u = plsc.load_gather(urows, [slotv, colv])
                v = plsc.load_gather(irows, [slotv, colv])
                acc = acc + u * v
            out_v[pl.ds(half * HALF + g * 16, 16)] = acc
            return 0

        lax.fori_loop(0, HALF // 16, group, 0)

    pltpu.sync_copy(out_v, out_hbm.at[wid])


@jax.jit
def _mf(users2, items2, ut, it):
    mesh = plsc.VectorSubcoreMesh(core_axis_name="c", subcore_axis_name="s")
    f = pl.kernel(
        _body,
        out_type=jax.ShapeDtypeStruct((NW, B_PER_W), jnp.float32),
        mesh=mesh,
        scratch_types=[
            pltpu.VMEM((B_PER_W + 16,), jnp.int32),
            pltpu.VMEM((B_PER_W + 16,), jnp.int32),
            pltpu.VMEM((HALF, N_FACTORS), jnp.float32),
            pltpu.VMEM((HALF, N_FACTORS), jnp.float32),
            pltpu.VMEM((B_PER_W,), jnp.float32),
            pltpu.SemaphoreType.DMA,
        ],
        compiler_params=pltpu.CompilerParams(needs_layout_passes=False),
    )
    return f(users2, items2, ut, it)


def kernel(users, items, user_table, item_table):
    users2 = users.reshape(NW, B_PER_W)
    items2 = items.reshape(NW, B_PER_W)
    out = _mf(users2, items2, user_table, item_table)
    return out.reshape(BATCH)
